# baseline (speedup 1.0000x reference)
import jax
import jax.numpy as jnp
from jax.experimental import pallas as pl
from jax.experimental.pallas import tpu as pltpu

_ROWS = 512
_COLS = 512
_KS = 5
_WI = _KS // 2
_NB = 4
_BR = _ROWS // _NB
_XR = _BR + 2 * _WI


def _splat_body(kw_ref, prev_ref, cur_ref, nxt_ref, o_ref, r_ref):
    b = pl.program_id(0)
    zt = jnp.zeros((_WI, _COLS), jnp.float32)
    top = jnp.where(b == 0, zt, prev_ref[_BR - _WI:, :])
    bot = jnp.where(b == _NB - 1, zt, nxt_ref[:_WI, :])
    x = jnp.concatenate([top, cur_ref[...], bot], axis=0)

    ras = [None] * _KS
    for v in range(_KS):
        d = v - _WI
        if d < 0:
            sv = jnp.concatenate(
                [jnp.zeros((_XR, -d), jnp.float32), x[:, :_COLS + d]], axis=1)
        elif d > 0:
            sv = jnp.concatenate(
                [x[:, d:], jnp.zeros((_XR, d), jnp.float32)], axis=1)
        else:
            sv = x
        kb = 2 * _WI - v
        for a in range(_KS):
            term = kw_ref[a, kb] * sv
            ras[a] = term if ras[a] is None else ras[a] + term
    for a in range(_KS):
        r_ref[a, :, :] = ras[a]
    acc = None
    for a in range(_KS):
        u = 2 * _WI - a
        term = r_ref[a, u:u + _BR, :]
        acc = term if acc is None else acc + term
    o_ref[...] = acc


def kernel(input, kernel):
    return pl.pallas_call(
        _splat_body,
        out_shape=jax.ShapeDtypeStruct((_ROWS, _COLS), input.dtype),
        grid=(_NB,),
        in_specs=[
            pl.BlockSpec(memory_space=pltpu.SMEM),
            pl.BlockSpec((_BR, _COLS), lambda i: (jnp.maximum(i - 1, 0), 0)),
            pl.BlockSpec((_BR, _COLS), lambda i: (i, 0)),
            pl.BlockSpec((_BR, _COLS),
                         lambda i: (jnp.minimum(i + 1, _NB - 1), 0)),
        ],
        out_specs=pl.BlockSpec((_BR, _COLS), lambda i: (i, 0)),
        scratch_shapes=[
            pltpu.VMEM((_KS, _XR, _COLS), jnp.float32),
        ],
    )(kernel, input, input, input)


# speedup vs baseline: 1.0369x; 1.0369x over previous
import jax
import jax.numpy as jnp
from jax.experimental import pallas as pl
from jax.experimental.pallas import tpu as pltpu

_ROWS = 512
_COLS = 512
_KS = 5
_WI = _KS // 2


def _splat_body(kw_ref, x_ref, o_ref, r_ref):
    x = x_ref[...]
    ras = [None] * _KS
    for v in range(_KS):
        d = v - _WI
        if d < 0:
            sv = jnp.concatenate(
                [jnp.zeros((_ROWS, -d), jnp.float32), x[:, :_COLS + d]], axis=1)
        elif d > 0:
            sv = jnp.concatenate(
                [x[:, d:], jnp.zeros((_ROWS, d), jnp.float32)], axis=1)
        else:
            sv = x
        b = 2 * _WI - v
        for a in range(_KS):
            term = kw_ref[a, b] * sv
            ras[a] = term if ras[a] is None else ras[a] + term
    for a in range(_KS):
        r_ref[a, :_WI, :] = jnp.zeros((_WI, _COLS), jnp.float32)
        r_ref[a, _WI:_WI + _ROWS, :] = ras[a]
        r_ref[a, _WI + _ROWS:, :] = jnp.zeros((_WI, _COLS), jnp.float32)
    acc = None
    for a in range(_KS):
        u = 2 * _WI - a
        term = r_ref[a, u:u + _ROWS, :]
        acc = term if acc is None else acc + term
    o_ref[...] = acc


def kernel(input, kernel):
    pad_rows = _ROWS + 2 * _WI
    return pl.pallas_call(
        _splat_body,
        out_shape=jax.ShapeDtypeStruct((_ROWS, _COLS), input.dtype),
        in_specs=[
            pl.BlockSpec(memory_space=pltpu.SMEM),
            pl.BlockSpec((_ROWS, _COLS), lambda: (0, 0)),
        ],
        out_specs=pl.BlockSpec((_ROWS, _COLS), lambda: (0, 0)),
        scratch_shapes=[
            pltpu.VMEM((_KS, pad_rows, _COLS), jnp.float32),
        ],
    )(kernel, input)
